# Initial kernel scaffold; baseline (speedup 1.0000x reference)
import jax
import jax.numpy as jnp
from jax.experimental import pallas as pl


def kernel(intermediate_states, w, full_topk_ids, full_topk_weight):
    raise NotImplementedError("write your pallas kernel here")


# speedup vs baseline: 1.2312x; 1.2312x over previous
import functools

import jax
import jax.numpy as jnp
from jax import lax
from jax.experimental import pallas as pl
from jax.experimental.pallas import tpu as pltpu
from jax.experimental.pallas import tpu_sc as plsc

HID = 1024
EXPERTS = 8
TOPK = 2
ROWS = 8192
TM = 256
S = ROWS + EXPERTS * TM
NW = 32


def _routing(ids, wt):
    oh = (ids[:, None] == jnp.arange(EXPERTS, dtype=ids.dtype)[None, :]).astype(jnp.int32)
    csum = jnp.cumsum(oh, axis=0)
    counts = csum[-1]
    rank = jnp.take_along_axis(csum, ids[:, None], axis=1)[:, 0] - 1
    padded = ((counts + TM - 1) // TM) * TM
    ends = jnp.cumsum(padded)
    offsets = ends - padded
    pos = offsets[ids] + rank
    inv = jnp.zeros((S,), jnp.int32).at[pos].set(jnp.arange(ROWS, dtype=jnp.int32))
    wsort = jnp.zeros((S,), jnp.float32).at[pos].set(wt)
    tile_starts = jnp.arange(S // TM, dtype=jnp.int32) * TM
    tile_expert = jnp.minimum(
        jnp.searchsorted(ends, tile_starts, side="right"), EXPERTS - 1
    ).astype(jnp.int32)
    return pos, inv, wsort, tile_expert


_G_CH = 64


def _sc_gather(table, idx):
    B = idx.shape[0]
    D = table.shape[1]
    b_per_w = B // NW
    n_ch = b_per_w // _G_CH
    mesh = plsc.VectorSubcoreMesh(core_axis_name="c", subcore_axis_name="s")

    @functools.partial(
        pl.kernel,
        mesh=mesh,
        out_type=jax.ShapeDtypeStruct((B, D), jnp.float32),
        scratch_types=[
            pltpu.VMEM((_G_CH,), jnp.int32),
            pltpu.VMEM((_G_CH, D), jnp.float32),
            pltpu.SemaphoreType.DMA,
        ],
    )
    def k(table_hbm, idx_hbm, out_hbm, idx_v, rows_v, sem):
        wid = lax.axis_index("s") * 2 + lax.axis_index("c")
        base = pl.multiple_of(wid * b_per_w, _G_CH)

        def body(c, _):
            off = pl.multiple_of(base + c * _G_CH, _G_CH)
            pltpu.sync_copy(idx_hbm.at[pl.ds(off, _G_CH)], idx_v)
            pltpu.async_copy(table_hbm.at[idx_v], rows_v, sem).wait()
            pltpu.sync_copy(rows_v, out_hbm.at[pl.ds(off, _G_CH)])
            return ()

        lax.fori_loop(0, n_ch, body, ())

    return k(table, idx)


_C_CH = 32


def _sc_combine(y, p0, p1):
    T = p0.shape[0]
    D = y.shape[1]
    t_per_w = T // NW
    n_ch = t_per_w // _C_CH
    nvec = _C_CH * D // 16
    mesh = plsc.VectorSubcoreMesh(core_axis_name="c", subcore_axis_name="s")

    @functools.partial(
        pl.kernel,
        mesh=mesh,
        out_type=jax.ShapeDtypeStruct((T, D), jnp.float32),
        scratch_types=[
            pltpu.VMEM((_C_CH,), jnp.int32),
            pltpu.VMEM((_C_CH,), jnp.int32),
            pltpu.VMEM((_C_CH, D), jnp.float32),
            pltpu.VMEM((_C_CH, D), jnp.float32),
            pltpu.SemaphoreType.DMA,
            pltpu.SemaphoreType.DMA,
        ],
    )
    def k(y_hbm, p0_hbm, p1_hbm, out_hbm, i0_v, i1_v, a_v, b_v, sem0, sem1):
        wid = lax.axis_index("s") * 2 + lax.axis_index("c")
        base = pl.multiple_of(wid * t_per_w, _C_CH)

        def body(c, _):
            off = pl.multiple_of(base + c * _C_CH, _C_CH)
            pltpu.sync_copy(p0_hbm.at[pl.ds(off, _C_CH)], i0_v)
            pltpu.sync_copy(p1_hbm.at[pl.ds(off, _C_CH)], i1_v)
            cp_a = pltpu.async_copy(y_hbm.at[i0_v], a_v, sem0)
            cp_b = pltpu.async_copy(y_hbm.at[i1_v], b_v, sem1)
            cp_a.wait()
            cp_b.wait()

            def add_body(i, _):
                r = i // (D // 16)
                j = (i % (D // 16)) * 16
                a_v[r, pl.ds(j, 16)] = a_v[r, pl.ds(j, 16)] + b_v[r, pl.ds(j, 16)]
                return ()

            lax.fori_loop(0, nvec, add_body, ())
            pltpu.sync_copy(a_v, out_hbm.at[pl.ds(off, _C_CH)])
            return ()

        lax.fori_loop(0, n_ch, body, ())

    return k(y, p0, p1)


def _gemm_body(te_ref, x_ref, w_ref, wv_ref, y_ref):
    x = x_ref[...].astype(jnp.bfloat16)
    y = jnp.dot(x, w_ref[0], preferred_element_type=jnp.float32)
    y_ref[...] = y * wv_ref[...]


def _grouped_gemm(x_sorted, w_bf, wsort, tile_expert):
    grid_spec = pltpu.PrefetchScalarGridSpec(
        num_scalar_prefetch=1,
        grid=(S // TM,),
        in_specs=[
            pl.BlockSpec((TM, HID), lambda i, te: (i, 0)),
            pl.BlockSpec((1, HID, HID), lambda i, te: (te[i], 0, 0)),
            pl.BlockSpec((TM, 1), lambda i, te: (i, 0)),
        ],
        out_specs=pl.BlockSpec((TM, HID), lambda i, te: (i, 0)),
    )
    return pl.pallas_call(
        _gemm_body,
        grid_spec=grid_spec,
        out_shape=jax.ShapeDtypeStruct((S, HID), jnp.float32),
    )(tile_expert, x_sorted, w_bf, wsort[:, None])


def kernel(intermediate_states, w, full_topk_ids, full_topk_weight):
    num_tokens = ROWS // TOPK
    ids = full_topk_ids[:num_tokens].reshape(-1)
    wt = full_topk_weight[:num_tokens].reshape(-1)

    pos, inv, wsort, tile_expert = _routing(ids, wt)

    x_sorted = _sc_gather(intermediate_states, inv)
    w_bf = w.astype(jnp.bfloat16)
    y_sorted = _grouped_gemm(x_sorted, w_bf, wsort, tile_expert)

    p0 = pos[0::2]
    p1 = pos[1::2]
    return _sc_combine(y_sorted, p0, p1)
